# baseline (speedup 1.0000x reference)
import functools
import math

import jax
import jax.numpy as jnp
from jax.experimental import pallas as pl
from jax.experimental.pallas import tpu as pltpu
from jax.experimental.pallas import tpu_sc as plsc

_ACT_SPARSITY = 0.65
_INT32_MIN = jnp.iinfo(jnp.int32).min
_INT32_MAX = jnp.iinfo(jnp.int32).max


def _topk_mask_kernel(x_ref, o_ref, *, k):
    x = x_ref[...]
    u = jax.lax.bitcast_convert_type(x, jnp.int32)
    key = jnp.where(u < 0, u ^ jnp.int32(0x7FFFFFFF), u)

    rows, n = x.shape
    plo0 = jnp.full((rows, 1), _INT32_MIN, dtype=jnp.int32)
    phi0 = jnp.full((rows, 1), _INT32_MAX, dtype=jnp.int32)

    def cond(carry):
        i, lo, hi = carry
        return (i < 32) & jnp.any(lo < hi)

    def body(carry):
        i, lo, hi = carry
        xor = lo ^ hi
        mid = (lo & hi) + (xor >> 1) + (xor & 1)
        cnt = jnp.sum((key >= mid).astype(jnp.int32), axis=1, keepdims=True)
        ge = cnt >= k
        eq = cnt == k
        lo = jnp.where(ge, mid, lo)
        hi = jnp.where(eq, mid, jnp.where(ge, hi, mid - 1))
        return i + 1, lo, hi

    _, lo, _ = jax.lax.while_loop(cond, body, (jnp.int32(0), plo0, phi0))
    o_ref[...] = jnp.where(key >= lo, x, 0.0)


def _tc_topk(x, k):
    m, n = x.shape
    block = 256
    while m % block:
        block //= 2
    grid = m // block
    return pl.pallas_call(
        functools.partial(_topk_mask_kernel, k=k),
        grid=(grid,),
        in_specs=[pl.BlockSpec((block, n), lambda i: (i, 0))],
        out_specs=pl.BlockSpec((block, n), lambda i: (i, 0)),
        out_shape=jax.ShapeDtypeStruct((m, n), x.dtype),
    )(x)


_NW = 32
_LANES = 16


def _sc_body(x_hbm, o_hbm, xbuf, kbuf, cbuf, hist, *, rows_per_worker, n, k,
             batch):
    iota = jax.lax.iota(jnp.int32, _LANES)
    ones = jnp.ones((_LANES,), jnp.int32)
    zeros16 = jnp.zeros((_LANES,), jnp.int32)
    ngr = n // _LANES

    wid = (jax.lax.axis_index("s") * 2 + jax.lax.axis_index("c")).astype(
        jnp.int32)
    row_base = wid * rows_per_worker

    def zero_hist():
        for b in range(32):
            hist[pl.ds(b * _LANES, _LANES)] = zeros16

    def scan_hist(k_rem):
        cum = jnp.int32(0)
        chosen = jnp.int32(0)
        above = jnp.int32(0)
        mcnt = jnp.int32(0)
        for b in range(31, -1, -1):
            s = jnp.sum(hist[pl.ds(b * _LANES, _LANES)])
            ncum = cum + s
            crossed = jnp.logical_and(cum < k_rem, ncum >= k_rem)
            chosen = jnp.where(crossed, jnp.int32(b), chosen)
            above = jnp.where(crossed, cum, above)
            mcnt = jnp.where(crossed, s, mcnt)
            cum = ncum
        return chosen, above, mcnt

    def row_body(rr, _):
        zero_hist()

        def l0(i, c):
            v = xbuf[rr, pl.ds(i * _LANES, _LANES)]
            bb = jax.lax.bitcast_convert_type(v, jnp.uint32)
            keyv = jnp.where(bb >= jnp.uint32(0x80000000), ~bb,
                             bb | jnp.uint32(0x80000000))
            kbuf[pl.ds(i * _LANES, _LANES)] = keyv
            digit = jax.lax.shift_right_logical(keyv, jnp.uint32(27))
            idx = jax.lax.bitcast_convert_type(digit << jnp.uint32(4), jnp.int32) + iota
            plsc.addupdate_scatter(hist, [idx], ones)
            return c

        jax.lax.fori_loop(0, ngr, l0, 0, unroll=4)

        chosen, above, mcnt = scan_hist(jnp.int32(k))
        chosen_u = jax.lax.convert_element_type(chosen, jnp.uint32)
        tkn = chosen_u << jnp.uint32(27)
        k_rem = jnp.int32(k) - above
        m_act = mcnt
        done = k_rem == m_act

        def c0(i, off):
            kv = kbuf[pl.ds(i * _LANES, _LANES)]
            digit = jax.lax.shift_right_logical(kv, jnp.uint32(27))
            msk = digit == chosen_u
            plsc.store_compressed(cbuf.at[pl.ds(off, _LANES)], kv, mask=msk)
            return off + jnp.sum(msk.astype(jnp.int32))

        jax.lax.fori_loop(0, ngr, c0, jnp.int32(0), unroll=4)

        def lcond(c):
            lvl, tkn, k_rem, m_act, done = c
            return jnp.logical_and(lvl < 7, jnp.logical_not(done))

        def lbody(c):
            lvl, tkn, k_rem, m_act, done = c
            sh = jnp.int32(27) - 5 * lvl
            sh_u = jax.lax.convert_element_type(
                jnp.maximum(sh, 0), jnp.uint32)
            wmask = jnp.where(sh >= 0, jnp.uint32(31), jnp.uint32(3))
            zero_hist()
            ngroups = (m_act + _LANES - 1) >> 4

            def hbody(i, c2):
                kv = cbuf[pl.ds(i * _LANES, _LANES)]
                valid = (i * _LANES + iota) < m_act
                digit = jax.lax.shift_right_logical(kv, sh_u) & wmask
                idx = jax.lax.bitcast_convert_type(digit << jnp.uint32(4), jnp.int32) + iota
                plsc.addupdate_scatter(hist, [idx], ones, mask=valid)
                return c2

            jax.lax.fori_loop(0, ngroups, hbody, 0)

            chosen, above, mcnt = scan_hist(k_rem)
            chosen_u = jax.lax.convert_element_type(chosen, jnp.uint32)
            tkn = tkn | (chosen_u << sh_u)
            k_rem = k_rem - above

            def cb(i, off):
                kv = cbuf[pl.ds(i * _LANES, _LANES)]
                valid = (i * _LANES + iota) < m_act
                digit = jax.lax.shift_right_logical(kv, sh_u) & wmask
                msk = jnp.logical_and(digit == chosen_u, valid)
                plsc.store_compressed(cbuf.at[pl.ds(off, _LANES)], kv,
                                      mask=msk)
                return off + jnp.sum(msk.astype(jnp.int32))

            jax.lax.fori_loop(0, ngroups, cb, jnp.int32(0))
            m_act = mcnt
            done = k_rem == m_act
            return lvl + 1, tkn, k_rem, m_act, done

        _, tkn, _, _, _ = jax.lax.while_loop(
            lcond, lbody, (jnp.int32(1), tkn, k_rem, m_act, done))

        def obody(i, c):
            kv = kbuf[pl.ds(i * _LANES, _LANES)]
            v = xbuf[rr, pl.ds(i * _LANES, _LANES)]
            xbuf[rr, pl.ds(i * _LANES, _LANES)] = jnp.where(
                kv >= tkn, v, jnp.float32(0.0))
            return c

        jax.lax.fori_loop(0, ngr, obody, 0, unroll=4)
        return 0

    def batch_body(bi, _):
        r0 = row_base + bi * batch
        pltpu.sync_copy(x_hbm.at[pl.ds(r0, batch)], xbuf)
        jax.lax.fori_loop(0, batch, row_body, 0)
        pltpu.sync_copy(xbuf, o_hbm.at[pl.ds(r0, batch)])
        return 0

    jax.lax.fori_loop(0, rows_per_worker // batch, batch_body, 0)


def _sc_topk(x, k):
    m, n = x.shape
    rows_per_worker = m // _NW
    batch = 8
    mesh = plsc.VectorSubcoreMesh(core_axis_name="c", subcore_axis_name="s")
    body = functools.partial(_sc_body, rows_per_worker=rows_per_worker,
                             n=n, k=k, batch=batch)
    return pl.kernel(
        body,
        out_type=jax.ShapeDtypeStruct((m, n), jnp.float32),
        mesh=mesh,
        compiler_params=pltpu.CompilerParams(needs_layout_passes=False),
        scratch_types=[
            pltpu.VMEM((batch, n), jnp.float32),
            pltpu.VMEM((n,), jnp.uint32),
            pltpu.VMEM((n + _LANES,), jnp.uint32),
            pltpu.VMEM((32 * _LANES,), jnp.int32),
        ],
    )(x)


def kernel(inputs):
    out_shape = inputs.shape
    x = inputs.reshape(inputs.shape[0], -1)
    m, n = x.shape
    k = math.floor((1.0 - _ACT_SPARSITY) * n)
    out = _sc_topk(x, k)
    return out.reshape(out_shape)


# speedup vs baseline: 3.7246x; 3.7246x over previous
import functools
import math

import jax
import jax.numpy as jnp
from jax.experimental import pallas as pl
from jax.experimental.pallas import tpu as pltpu
from jax.experimental.pallas import tpu_sc as plsc

_ACT_SPARSITY = 0.65
_INT32_MIN = jnp.iinfo(jnp.int32).min
_INT32_MAX = jnp.iinfo(jnp.int32).max


def _topk_mask_kernel(x_ref, o_ref, *, k):
    x = x_ref[...]
    u = jax.lax.bitcast_convert_type(x, jnp.int32)
    key = jnp.where(u < 0, u ^ jnp.int32(0x7FFFFFFF), u)

    rows, n = x.shape
    plo0 = jnp.full((rows, 1), _INT32_MIN, dtype=jnp.int32)
    phi0 = jnp.full((rows, 1), _INT32_MAX, dtype=jnp.int32)

    def cond(carry):
        i, lo, hi = carry
        return (i < 32) & jnp.any(lo < hi)

    def body(carry):
        i, lo, hi = carry
        xor = lo ^ hi
        mid = (lo & hi) + (xor >> 1) + (xor & 1)
        cnt = jnp.sum((key >= mid).astype(jnp.int32), axis=1, keepdims=True)
        ge = cnt >= k
        eq = cnt == k
        lo = jnp.where(ge, mid, lo)
        hi = jnp.where(eq, mid, jnp.where(ge, hi, mid - 1))
        return i + 1, lo, hi

    _, lo, _ = jax.lax.while_loop(cond, body, (jnp.int32(0), plo0, phi0))
    o_ref[...] = jnp.where(key >= lo, x, 0.0)


def _tc_topk(x, k):
    m, n = x.shape
    block = 256
    while m % block:
        block //= 2
    grid = m // block
    return pl.pallas_call(
        functools.partial(_topk_mask_kernel, k=k),
        grid=(grid,),
        in_specs=[pl.BlockSpec((block, n), lambda i: (i, 0))],
        out_specs=pl.BlockSpec((block, n), lambda i: (i, 0)),
        out_shape=jax.ShapeDtypeStruct((m, n), x.dtype),
    )(x)


_NW = 32
_LANES = 16


def _sc_body(x_hbm, o_hbm, xbuf, kbuf, cbuf, hist, *, rows_per_worker, n, k,
             batch):
    iota = jax.lax.iota(jnp.int32, _LANES)
    ones = jnp.ones((_LANES,), jnp.int32)
    zeros16 = jnp.zeros((_LANES,), jnp.int32)
    ngr = n // _LANES

    wid = (jax.lax.axis_index("s") * 2 + jax.lax.axis_index("c")).astype(
        jnp.int32)
    row_base = wid * rows_per_worker

    def zero_hist():
        for b in range(32):
            hist[pl.ds(b * _LANES, _LANES)] = zeros16

    def scan_hist(k_rem):
        cum = jnp.int32(0)
        chosen = jnp.int32(0)
        above = jnp.int32(0)
        mcnt = jnp.int32(0)
        for b in range(31, -1, -1):
            s = jnp.sum(hist[pl.ds(b * _LANES, _LANES)])
            ncum = cum + s
            crossed = jnp.logical_and(cum < k_rem, ncum >= k_rem)
            chosen = jnp.where(crossed, jnp.int32(b), chosen)
            above = jnp.where(crossed, cum, above)
            mcnt = jnp.where(crossed, s, mcnt)
            cum = ncum
        return chosen, above, mcnt

    def row_body(rr, _):
        zero_hist()

        def l0(i, c):
            v = xbuf[rr, pl.ds(i * _LANES, _LANES)]
            bb = jax.lax.bitcast_convert_type(v, jnp.uint32)
            keyv = jnp.where(bb >= jnp.uint32(0x80000000), ~bb,
                             bb | jnp.uint32(0x80000000))
            kbuf[pl.ds(i * _LANES, _LANES)] = keyv
            digit = jax.lax.shift_right_logical(keyv, jnp.uint32(27))
            idx = jax.lax.bitcast_convert_type(digit << jnp.uint32(4), jnp.int32) + iota
            plsc.addupdate_scatter(hist, [idx], ones)
            return c

        jax.lax.fori_loop(0, ngr, l0, 0, unroll=4)

        chosen, above, mcnt = scan_hist(jnp.int32(k))
        chosen_u = jax.lax.convert_element_type(chosen, jnp.uint32)
        tkn = chosen_u << jnp.uint32(27)
        k_rem = jnp.int32(k) - above
        m_act = mcnt
        done = k_rem == m_act

        def c0(i, off):
            kv = kbuf[pl.ds(i * _LANES, _LANES)]
            digit = jax.lax.shift_right_logical(kv, jnp.uint32(27))
            msk = digit == chosen_u
            plsc.store_compressed(cbuf.at[pl.ds(off, _LANES)], kv, mask=msk)
            return off + jnp.sum(msk.astype(jnp.int32))

        jax.lax.fori_loop(0, ngr, c0, jnp.int32(0), unroll=4)

        def lcond(c):
            lvl, tkn, k_rem, m_act, done = c
            return jnp.logical_and(lvl < 7, jnp.logical_not(done))

        def lbody(c):
            lvl, tkn, k_rem, m_act, done = c
            sh = jnp.int32(27) - 5 * lvl
            sh_u = jax.lax.convert_element_type(
                jnp.maximum(sh, 0), jnp.uint32)
            wmask = jnp.where(sh >= 0, jnp.uint32(31), jnp.uint32(3))
            zero_hist()
            ngroups = (m_act + _LANES - 1) >> 4

            def hbody(i, c2):
                kv = cbuf[pl.ds(i * _LANES, _LANES)]
                valid = (i * _LANES + iota) < m_act
                digit = jax.lax.shift_right_logical(kv, sh_u) & wmask
                idx = jax.lax.bitcast_convert_type(digit << jnp.uint32(4), jnp.int32) + iota
                plsc.addupdate_scatter(hist, [idx], ones, mask=valid)
                return c2

            jax.lax.fori_loop(0, ngroups, hbody, 0)

            chosen, above, mcnt = scan_hist(k_rem)
            chosen_u = jax.lax.convert_element_type(chosen, jnp.uint32)
            tkn = tkn | (chosen_u << sh_u)
            k_rem = k_rem - above

            def cb(i, off):
                kv = cbuf[pl.ds(i * _LANES, _LANES)]
                valid = (i * _LANES + iota) < m_act
                digit = jax.lax.shift_right_logical(kv, sh_u) & wmask
                msk = jnp.logical_and(digit == chosen_u, valid)
                plsc.store_compressed(cbuf.at[pl.ds(off, _LANES)], kv,
                                      mask=msk)
                return off + jnp.sum(msk.astype(jnp.int32))

            jax.lax.fori_loop(0, ngroups, cb, jnp.int32(0))
            m_act = mcnt
            done = k_rem == m_act
            return lvl + 1, tkn, k_rem, m_act, done

        _, tkn, _, _, _ = jax.lax.while_loop(
            lcond, lbody, (jnp.int32(1), tkn, k_rem, m_act, done))

        def obody(i, c):
            kv = kbuf[pl.ds(i * _LANES, _LANES)]
            v = xbuf[rr, pl.ds(i * _LANES, _LANES)]
            xbuf[rr, pl.ds(i * _LANES, _LANES)] = jnp.where(
                kv >= tkn, v, jnp.float32(0.0))
            return c

        jax.lax.fori_loop(0, ngr, obody, 0, unroll=4)
        return 0

    def batch_body(bi, _):
        r0 = row_base + bi * batch
        pltpu.sync_copy(x_hbm.at[pl.ds(r0, batch)], xbuf)
        jax.lax.fori_loop(0, batch, row_body, 0)
        pltpu.sync_copy(xbuf, o_hbm.at[pl.ds(r0, batch)])
        return 0

    jax.lax.fori_loop(0, rows_per_worker // batch, batch_body, 0)


def _sc_topk(x, k):
    m, n = x.shape
    rows_per_worker = m // _NW
    batch = 8
    mesh = plsc.VectorSubcoreMesh(core_axis_name="c", subcore_axis_name="s")
    body = functools.partial(_sc_body, rows_per_worker=rows_per_worker,
                             n=n, k=k, batch=batch)
    return pl.kernel(
        body,
        out_type=jax.ShapeDtypeStruct((m, n), jnp.float32),
        mesh=mesh,
        compiler_params=pltpu.CompilerParams(needs_layout_passes=False),
        scratch_types=[
            pltpu.VMEM((batch, n), jnp.float32),
            pltpu.VMEM((n,), jnp.uint32),
            pltpu.VMEM((n + _LANES,), jnp.uint32),
            pltpu.VMEM((32 * _LANES,), jnp.int32),
        ],
    )(x)


def kernel(inputs):
    out_shape = inputs.shape
    x = inputs.reshape(inputs.shape[0], -1)
    m, n = x.shape
    k = math.floor((1.0 - _ACT_SPARSITY) * n)
    m_sc = (m * 13 // 64) // _NW * _NW
    while m_sc and (m_sc // _NW) % 8:
        m_sc -= _NW
    m_tc = m - m_sc
    if m_sc == 0:
        out = _tc_topk(x, k)
    else:
        out_tc = _tc_topk(x[:m_tc], k)
        out_sc = _sc_topk(x[m_tc:], k)
        out = jnp.concatenate([out_tc, out_sc], axis=0)
    return out.reshape(out_shape)


# speedup vs baseline: 4.6167x; 1.2395x over previous
import functools
import math

import jax
import jax.numpy as jnp
from jax.experimental import pallas as pl
from jax.experimental.pallas import tpu as pltpu
from jax.experimental.pallas import tpu_sc as plsc

_ACT_SPARSITY = 0.65
_INT32_MIN = jnp.iinfo(jnp.int32).min
_INT32_MAX = jnp.iinfo(jnp.int32).max


def _topk_mask_kernel(x_ref, o_ref, *, k):
    x = x_ref[...]
    u = jax.lax.bitcast_convert_type(x, jnp.int32)
    key = jnp.where(u < 0, u ^ jnp.int32(0x7FFFFFFF), u)

    rows, n = x.shape
    plo0 = jnp.full((rows, 1), _INT32_MIN, dtype=jnp.int32)
    phi0 = jnp.full((rows, 1), _INT32_MAX, dtype=jnp.int32)

    def cond(carry):
        i, lo, hi = carry
        return (i < 32) & jnp.any(lo < hi)

    def body(carry):
        i, lo, hi = carry
        xor = lo ^ hi
        mid = (lo & hi) + (xor >> 1) + (xor & 1)
        cnt = jnp.sum((key >= mid).astype(jnp.int32), axis=1, keepdims=True)
        ge = cnt >= k
        eq = cnt == k
        lo = jnp.where(ge, mid, lo)
        hi = jnp.where(eq, mid, jnp.where(ge, hi, mid - 1))
        return i + 1, lo, hi

    _, lo, _ = jax.lax.while_loop(cond, body, (jnp.int32(0), plo0, phi0))
    o_ref[...] = jnp.where(key >= lo, x, 0.0)


def _tc_topk(x, k):
    m, n = x.shape
    block = 256
    while m % block:
        block //= 2
    grid = m // block
    return pl.pallas_call(
        functools.partial(_topk_mask_kernel, k=k),
        grid=(grid,),
        in_specs=[pl.BlockSpec((block, n), lambda i: (i, 0))],
        out_specs=pl.BlockSpec((block, n), lambda i: (i, 0)),
        out_shape=jax.ShapeDtypeStruct((m, n), x.dtype),
    )(x)


_NW = 32
_LANES = 16


def _sc_body(x_hbm, o_hbm, xbuf, kbuf, cbuf, hist, *, rows_per_worker, n, k,
             batch, in_row_off):
    iota = jax.lax.iota(jnp.int32, _LANES)
    ones = jnp.ones((_LANES,), jnp.int32)
    zeros16 = jnp.zeros((_LANES,), jnp.int32)
    ngr = n // _LANES

    wid = (jax.lax.axis_index("s") * 2 + jax.lax.axis_index("c")).astype(
        jnp.int32)
    row_base = wid * rows_per_worker

    def zero_hist():
        for b in range(32):
            hist[pl.ds(b * _LANES, _LANES)] = zeros16

    def scan_hist(k_rem):
        cum = jnp.int32(0)
        chosen = jnp.int32(0)
        above = jnp.int32(0)
        mcnt = jnp.int32(0)
        for b in range(31, -1, -1):
            s = jnp.sum(hist[pl.ds(b * _LANES, _LANES)])
            ncum = cum + s
            crossed = jnp.logical_and(cum < k_rem, ncum >= k_rem)
            chosen = jnp.where(crossed, jnp.int32(b), chosen)
            above = jnp.where(crossed, cum, above)
            mcnt = jnp.where(crossed, s, mcnt)
            cum = ncum
        return chosen, above, mcnt

    def row_body(rr, _):
        zero_hist()

        def l0(i, c):
            v = xbuf[rr, pl.ds(i * _LANES, _LANES)]
            bb = jax.lax.bitcast_convert_type(v, jnp.uint32)
            keyv = jnp.where(bb >= jnp.uint32(0x80000000), ~bb,
                             bb | jnp.uint32(0x80000000))
            kbuf[pl.ds(i * _LANES, _LANES)] = keyv
            digit = jax.lax.shift_right_logical(keyv, jnp.uint32(27))
            idx = jax.lax.bitcast_convert_type(digit << jnp.uint32(4), jnp.int32) + iota
            plsc.addupdate_scatter(hist, [idx], ones)
            return c

        jax.lax.fori_loop(0, ngr, l0, 0, unroll=4)

        chosen, above, mcnt = scan_hist(jnp.int32(k))
        chosen_u = jax.lax.convert_element_type(chosen, jnp.uint32)
        tkn = chosen_u << jnp.uint32(27)
        k_rem = jnp.int32(k) - above
        m_act = mcnt
        done = k_rem == m_act

        def c0(i, off):
            kv = kbuf[pl.ds(i * _LANES, _LANES)]
            digit = jax.lax.shift_right_logical(kv, jnp.uint32(27))
            msk = digit == chosen_u
            plsc.store_compressed(cbuf.at[pl.ds(off, _LANES)], kv, mask=msk)
            return off + jnp.sum(msk.astype(jnp.int32))

        jax.lax.fori_loop(0, ngr, c0, jnp.int32(0), unroll=4)

        def lcond(c):
            lvl, tkn, k_rem, m_act, done = c
            return jnp.logical_and(lvl < 7, jnp.logical_not(done))

        def lbody(c):
            lvl, tkn, k_rem, m_act, done = c
            sh = jnp.int32(27) - 5 * lvl
            sh_u = jax.lax.convert_element_type(
                jnp.maximum(sh, 0), jnp.uint32)
            wmask = jnp.where(sh >= 0, jnp.uint32(31), jnp.uint32(3))
            zero_hist()
            ngroups = (m_act + _LANES - 1) >> 4

            def hbody(i, c2):
                kv = cbuf[pl.ds(i * _LANES, _LANES)]
                valid = (i * _LANES + iota) < m_act
                digit = jax.lax.shift_right_logical(kv, sh_u) & wmask
                idx = jax.lax.bitcast_convert_type(digit << jnp.uint32(4), jnp.int32) + iota
                plsc.addupdate_scatter(hist, [idx], ones, mask=valid)
                return c2

            jax.lax.fori_loop(0, ngroups, hbody, 0)

            chosen, above, mcnt = scan_hist(k_rem)
            chosen_u = jax.lax.convert_element_type(chosen, jnp.uint32)
            tkn = tkn | (chosen_u << sh_u)
            k_rem = k_rem - above

            def cb(i, off):
                kv = cbuf[pl.ds(i * _LANES, _LANES)]
                valid = (i * _LANES + iota) < m_act
                digit = jax.lax.shift_right_logical(kv, sh_u) & wmask
                msk = jnp.logical_and(digit == chosen_u, valid)
                plsc.store_compressed(cbuf.at[pl.ds(off, _LANES)], kv,
                                      mask=msk)
                return off + jnp.sum(msk.astype(jnp.int32))

            jax.lax.fori_loop(0, ngroups, cb, jnp.int32(0))
            m_act = mcnt
            done = k_rem == m_act
            return lvl + 1, tkn, k_rem, m_act, done

        _, tkn, _, _, _ = jax.lax.while_loop(
            lcond, lbody, (jnp.int32(1), tkn, k_rem, m_act, done))

        def obody(i, c):
            kv = kbuf[pl.ds(i * _LANES, _LANES)]
            v = xbuf[rr, pl.ds(i * _LANES, _LANES)]
            xbuf[rr, pl.ds(i * _LANES, _LANES)] = jnp.where(
                kv >= tkn, v, jnp.float32(0.0))
            return c

        jax.lax.fori_loop(0, ngr, obody, 0, unroll=4)
        return 0

    def batch_body(bi, _):
        r0 = row_base + bi * batch
        pltpu.sync_copy(x_hbm.at[pl.ds(r0 + in_row_off, batch)], xbuf)
        jax.lax.fori_loop(0, batch, row_body, 0)
        pltpu.sync_copy(xbuf, o_hbm.at[pl.ds(r0, batch)])
        return 0

    jax.lax.fori_loop(0, rows_per_worker // batch, batch_body, 0)


def _sc_topk(x, k, row_off, m_sc):
    m, n = x.shape
    rows_per_worker = m_sc // _NW
    batch = 8
    mesh = plsc.VectorSubcoreMesh(core_axis_name="c", subcore_axis_name="s")
    body = functools.partial(_sc_body, rows_per_worker=rows_per_worker,
                             n=n, k=k, batch=batch, in_row_off=row_off)
    return pl.kernel(
        body,
        out_type=jax.ShapeDtypeStruct((m_sc, n), jnp.float32),
        mesh=mesh,
        compiler_params=pltpu.CompilerParams(needs_layout_passes=False),
        scratch_types=[
            pltpu.VMEM((batch, n), jnp.float32),
            pltpu.VMEM((n,), jnp.uint32),
            pltpu.VMEM((n + _LANES,), jnp.uint32),
            pltpu.VMEM((32 * _LANES,), jnp.int32),
        ],
    )(x)


def kernel(inputs):
    out_shape = inputs.shape
    x = inputs.reshape(inputs.shape[0], -1)
    m, n = x.shape
    k = math.floor((1.0 - _ACT_SPARSITY) * n)
    m_sc = (m * 13 // 64) // _NW * _NW
    while m_sc and ((m_sc // _NW) % 8 or (m - m_sc) % 256):
        m_sc -= _NW
    m_tc = m - m_sc
    if m_sc == 0:
        out = _tc_topk(x, k)
    else:
        block = 256
        out_tc = pl.pallas_call(
            functools.partial(_topk_mask_kernel, k=k),
            grid=(m_tc // block,),
            in_specs=[pl.BlockSpec((block, n), lambda i: (i, 0))],
            out_specs=pl.BlockSpec((block, n), lambda i: (i, 0)),
            out_shape=jax.ShapeDtypeStruct((m, n), x.dtype),
        )(x)
        out_sc = _sc_topk(x, k, m_tc, m_sc)
        out = jax.lax.dynamic_update_slice(out_tc, out_sc, (m_tc, 0))
    return out.reshape(out_shape)
